# Initial kernel scaffold; baseline (speedup 1.0000x reference)
import jax
import jax.numpy as jnp
from jax.experimental import pallas as pl


def kernel(w, E, swr):
    raise NotImplementedError("write your pallas kernel here")


# speedup vs baseline: 7.4917x; 7.4917x over previous
import jax
import jax.numpy as jnp
from jax.experimental import pallas as pl
from jax.experimental.pallas import tpu as pltpu

_B = 128
_N_RODT = 512
_N_EST = 64
_N_FOREST = 100
_F_PAD = 104
_N_HIDDEN = 128


def _forest_kernel(swr_ref, w_ref, e_ref, out_ref, m_ref):
    b = pl.program_id(0)

    @pl.when(b == 0)
    def _build_m():
        sw = swr_ref[0]
        iota = jax.lax.broadcasted_iota(jnp.int32, (_F_PAD, _N_RODT), 1)
        m = jnp.zeros((_F_PAD, _N_RODT), jnp.float32)
        for e in range(_N_EST):
            col = jax.lax.slice(sw, (0, e), (_F_PAD, e + 1))
            m = m + (col == iota).astype(jnp.float32)
        m_ref[...] = m

    wrow = w_ref[0]
    ew = jnp.exp(wrow - jnp.max(wrow))
    a = m_ref[...] * ew
    d = jnp.sum(a, axis=1, keepdims=True)
    n = jnp.dot(a, e_ref[0], preferred_element_type=jnp.float32,
                precision=jax.lax.Precision.HIGHEST)
    out_ref[0] = n[:_N_FOREST] / d[:_N_FOREST]


def kernel(w, E, swr):
    swr_pad = jnp.pad(swr.astype(jnp.int32),
                      ((0, _F_PAD - _N_FOREST), (0, 0)),
                      constant_values=-1)[None]
    w3 = w.reshape(_B, 1, _N_RODT)
    return pl.pallas_call(
        _forest_kernel,
        grid=(_B,),
        in_specs=[
            pl.BlockSpec((1, _F_PAD, _N_EST), lambda b: (0, 0, 0)),
            pl.BlockSpec((1, 1, _N_RODT), lambda b: (b, 0, 0)),
            pl.BlockSpec((1, _N_RODT, _N_HIDDEN), lambda b: (b, 0, 0)),
        ],
        out_specs=pl.BlockSpec((1, _N_FOREST, _N_HIDDEN), lambda b: (b, 0, 0)),
        out_shape=jax.ShapeDtypeStruct((_B, _N_FOREST, _N_HIDDEN), jnp.float32),
        scratch_shapes=[pltpu.VMEM((_F_PAD, _N_RODT), jnp.float32)],
    )(swr_pad, w3, E)


# speedup vs baseline: 18.1290x; 2.4199x over previous
import jax
import jax.numpy as jnp
from jax.experimental import pallas as pl
from jax.experimental.pallas import tpu as pltpu

_B = 128
_N_RODT = 512
_N_EST = 64
_N_FOREST = 100
_F_PAD = 104
_N_HIDDEN = 128


_BB = 8


def _forest_kernel(swr_ref, w_ref, e_ref, out_ref, m_ref):
    b = pl.program_id(0)

    @pl.when(b == 0)
    def _build_m():
        sw = swr_ref[0]
        iota = jax.lax.broadcasted_iota(jnp.int32, (_F_PAD, _N_RODT), 1)
        m = jnp.zeros((_F_PAD, _N_RODT), jnp.float32)
        for e in range(_N_EST):
            col = jax.lax.slice(sw, (0, e), (_F_PAD, e + 1))
            m = m + (col == iota).astype(jnp.float32)
        m_ref[...] = m

    for bb in range(_BB):
        wrow = w_ref[bb]
        ew = jnp.exp(wrow - jnp.max(wrow))
        a = m_ref[...] * ew
        d = jnp.sum(a, axis=1, keepdims=True)
        n = jnp.dot(a, e_ref[bb], preferred_element_type=jnp.float32,
                    precision=jax.lax.Precision.HIGHEST)
        out_ref[bb] = n[:_N_FOREST] / d[:_N_FOREST]


def kernel(w, E, swr):
    swr_pad = jnp.pad(swr.astype(jnp.int32),
                      ((0, _F_PAD - _N_FOREST), (0, 0)),
                      constant_values=-1)[None]
    w3 = w.reshape(_B, 1, _N_RODT)
    return pl.pallas_call(
        _forest_kernel,
        grid=(_B // _BB,),
        in_specs=[
            pl.BlockSpec((1, _F_PAD, _N_EST), lambda b: (0, 0, 0)),
            pl.BlockSpec((_BB, 1, _N_RODT), lambda b: (b, 0, 0)),
            pl.BlockSpec((_BB, _N_RODT, _N_HIDDEN), lambda b: (b, 0, 0)),
        ],
        out_specs=pl.BlockSpec((_BB, _N_FOREST, _N_HIDDEN), lambda b: (b, 0, 0)),
        out_shape=jax.ShapeDtypeStruct((_B, _N_FOREST, _N_HIDDEN), jnp.float32),
        scratch_shapes=[pltpu.VMEM((_F_PAD, _N_RODT), jnp.float32)],
    )(swr_pad, w3, E)


# speedup vs baseline: 25.0882x; 1.3839x over previous
import jax
import jax.numpy as jnp
from jax.experimental import pallas as pl
from jax.experimental.pallas import tpu as pltpu

_B = 128
_N_RODT = 512
_N_EST = 64
_N_FOREST = 100
_F_PAD = 104
_N_HIDDEN = 128


_BB = 8


def _forest_kernel(swr_ref, w_ref, e_ref, out_ref, m_ref):
    b = pl.program_id(0)

    @pl.when(b == 0)
    def _build_m():
        sw = swr_ref[0]
        iota = jax.lax.broadcasted_iota(jnp.int32, (_F_PAD, _N_RODT), 1)
        m = jnp.zeros((_F_PAD, _N_RODT), jnp.float32)
        for e in range(_N_EST):
            col = jax.lax.slice(sw, (0, e), (_F_PAD, e + 1))
            m = m + (col == iota).astype(jnp.float32)
        m_ref[...] = m

    for bb in range(_BB):
        wrow = w_ref[bb]
        ew = jnp.exp(wrow - jnp.max(wrow))
        a = m_ref[...] * ew
        d = jnp.sum(a, axis=1, keepdims=True)
        n = jnp.dot(a, e_ref[bb], preferred_element_type=jnp.float32,
                    precision=jax.lax.Precision.DEFAULT)
        out_ref[bb] = n[:_N_FOREST] * (1.0 / d[:_N_FOREST])


def kernel(w, E, swr):
    swr_pad = jnp.pad(swr.astype(jnp.int32),
                      ((0, _F_PAD - _N_FOREST), (0, 0)),
                      constant_values=-1)[None]
    w3 = w.reshape(_B, 1, _N_RODT)
    return pl.pallas_call(
        _forest_kernel,
        grid=(_B // _BB,),
        in_specs=[
            pl.BlockSpec((1, _F_PAD, _N_EST), lambda b: (0, 0, 0)),
            pl.BlockSpec((_BB, 1, _N_RODT), lambda b: (b, 0, 0)),
            pl.BlockSpec((_BB, _N_RODT, _N_HIDDEN), lambda b: (b, 0, 0)),
        ],
        out_specs=pl.BlockSpec((_BB, _N_FOREST, _N_HIDDEN), lambda b: (b, 0, 0)),
        out_shape=jax.ShapeDtypeStruct((_B, _N_FOREST, _N_HIDDEN), jnp.float32),
        scratch_shapes=[pltpu.VMEM((_F_PAD, _N_RODT), jnp.float32)],
    )(swr_pad, w3, E)
